# Initial kernel scaffold; baseline (speedup 1.0000x reference)
import jax
import jax.numpy as jnp
from jax.experimental import pallas as pl


def kernel(src, dates, encoding):
    raise NotImplementedError("write your pallas kernel here")


# speedup vs baseline: 2.0568x; 2.0568x over previous
import functools

import jax
import jax.numpy as jnp
from jax import lax
from jax.experimental import pallas as pl
from jax.experimental.pallas import tpu as pltpu
from jax.experimental.pallas import tpu_sc as plsc

ROWS = 12
COLS = 31
LANES = 16


@functools.lru_cache(maxsize=None)
def _build_sc_kernel(n_tokens, d, t_chunk):
    info = plsc.get_sparse_core_info()
    nc, ns = info.num_cores, info.num_subcores
    nw = nc * ns
    per_w = n_tokens // nw
    n_chunks = per_w // t_chunk
    mesh = plsc.VectorSubcoreMesh(core_axis_name="c", subcore_axis_name="s")

    @functools.partial(
        pl.kernel,
        mesh=mesh,
        out_type=jax.ShapeDtypeStruct((n_tokens, d), jnp.float32),
        scratch_types=[
            pltpu.VMEM((t_chunk,), jnp.int32),
            pltpu.VMEM((t_chunk,), jnp.int32),
            pltpu.VMEM((t_chunk,), jnp.int32),
            pltpu.VMEM((t_chunk, d), jnp.float32),
            pltpu.VMEM((t_chunk, d), jnp.float32),
            pltpu.SemaphoreType.DMA,
            pltpu.SemaphoreType.DMA,
        ],
    )
    def k(src_hbm, r_hbm, c_hbm, table_hbm, out_hbm,
          r_v, c_v, idx_v, src_v, enc_v, sem_a, sem_b):
        wid = lax.axis_index("s") * nc + lax.axis_index("c")
        base = wid * per_w

        def chunk_body(i, carry):
            off = base + i * t_chunk
            pltpu.sync_copy(r_hbm.at[pl.ds(off, t_chunk)], r_v)
            pltpu.sync_copy(c_hbm.at[pl.ds(off, t_chunk)], c_v)
            for j in range(t_chunk // LANES):
                sl = pl.ds(j * LANES, LANES)
                rv = r_v[sl] - 1
                rv = jnp.where(rv < 0, rv + ROWS, rv)
                cv = c_v[sl] - 1
                cv = jnp.where(cv < 0, cv + COLS, cv)
                idx_v[sl] = rv * COLS + cv
            cp_s = pltpu.async_copy(src_hbm.at[pl.ds(off, t_chunk)], src_v, sem_a)
            cp_g = pltpu.async_copy(table_hbm.at[idx_v], enc_v, sem_b)
            cp_s.wait()
            cp_g.wait()

            def add_row(t, c2):
                for j in range(d // LANES):
                    sl = pl.ds(j * LANES, LANES)
                    src_v[t, sl] = src_v[t, sl] + enc_v[t, sl]
                return c2

            lax.fori_loop(0, t_chunk, add_row, 0)
            pltpu.sync_copy(src_v, out_hbm.at[pl.ds(off, t_chunk)])
            return carry

        lax.fori_loop(0, n_chunks, chunk_body, 0)

    return k


def kernel(src, dates, encoding):
    b, s, d = src.shape
    n = b * s
    src2 = src.reshape(n, d)
    r = dates[..., 0].astype(jnp.int32).reshape(n)
    c = dates[..., 1].astype(jnp.int32).reshape(n)
    table = encoding.reshape(-1, d)
    out = _build_sc_kernel(n, d, 32)(src2, r, c, table)
    return out.reshape(b, s, d)


# speedup vs baseline: 2.1440x; 1.0424x over previous
import functools

import jax
import jax.numpy as jnp
from jax import lax
from jax.experimental import pallas as pl
from jax.experimental.pallas import tpu as pltpu
from jax.experimental.pallas import tpu_sc as plsc

ROWS = 12
COLS = 31
LANES = 16


@functools.lru_cache(maxsize=None)
def _build_sc_kernel(n_tokens, d, t_chunk):
    info = plsc.get_sparse_core_info()
    nc, ns = info.num_cores, info.num_subcores
    nw = nc * ns
    per_w = n_tokens // nw
    n_chunks = per_w // t_chunk
    n_pairs = n_chunks // 2
    n_ivec = per_w // LANES
    mesh = plsc.VectorSubcoreMesh(core_axis_name="c", subcore_axis_name="s")

    @functools.partial(
        pl.kernel,
        mesh=mesh,
        out_type=jax.ShapeDtypeStruct((n_tokens, d), jnp.float32),
        scratch_types=[
            pltpu.VMEM((per_w,), jnp.int32),
            pltpu.VMEM((per_w,), jnp.int32),
            pltpu.VMEM((per_w,), jnp.int32),
            pltpu.VMEM((t_chunk, d), jnp.float32),
            pltpu.VMEM((t_chunk, d), jnp.float32),
            pltpu.VMEM((t_chunk, d), jnp.float32),
            pltpu.VMEM((t_chunk, d), jnp.float32),
            pltpu.SemaphoreType.DMA,
            pltpu.SemaphoreType.DMA,
            pltpu.SemaphoreType.DMA,
            pltpu.SemaphoreType.DMA,
            pltpu.SemaphoreType.DMA,
            pltpu.SemaphoreType.DMA,
        ],
    )
    def k(src_hbm, r_hbm, c_hbm, table_hbm, out_hbm,
          r_v, c_v, idx_v, src0, src1, enc0, enc1,
          sem_s0, sem_s1, sem_g0, sem_g1, sem_o0, sem_o1):
        wid = lax.axis_index("s") * nc + lax.axis_index("c")
        base = wid * per_w
        srcs = (src0, src1)
        encs = (enc0, enc1)
        sems_s = (sem_s0, sem_s1)
        sems_g = (sem_g0, sem_g1)
        sems_o = (sem_o0, sem_o1)

        pltpu.sync_copy(r_hbm.at[pl.ds(base, per_w)], r_v)
        pltpu.sync_copy(c_hbm.at[pl.ds(base, per_w)], c_v)

        def idx_body(u, carry):
            sl = pl.ds(u * LANES, LANES)
            rv = r_v[sl] - 1
            rv = jnp.where(rv < 0, rv + ROWS, rv)
            cv = c_v[sl] - 1
            cv = jnp.where(cv < 0, cv + COLS, cv)
            idx_v[sl] = rv * COLS + cv
            return carry

        lax.fori_loop(0, n_ivec, idx_body, 0)

        def in_copies(ci, b):
            off = base + ci * t_chunk
            cs = pltpu.make_async_copy(
                src_hbm.at[pl.ds(off, t_chunk)], srcs[b], sems_s[b])
            cg = pltpu.make_async_copy(
                table_hbm.at[idx_v.at[pl.ds(ci * t_chunk, t_chunk)]],
                encs[b], sems_g[b])
            return cs, cg

        def issue_in(ci, b):
            cs, cg = in_copies(ci, b)
            cs.start()
            cg.start()

        def wait_in(ci, b):
            cs, cg = in_copies(ci, b)
            cs.wait()
            cg.wait()

        def add_chunk(b):
            def body(t, carry):
                for j in range(d // LANES):
                    sl = pl.ds(j * LANES, LANES)
                    plsc.addupdate(srcs[b].at[t, sl], encs[b][t, sl])
                return carry

            lax.fori_loop(0, t_chunk, body, 0)

        issue_in(0, 0)
        issue_in(1, 1)

        def pair_body(g, carry):
            c0 = 2 * g
            c1 = 2 * g + 1
            wait_in(c0, 0)
            add_chunk(0)
            out0 = pltpu.async_copy(
                srcs[0], out_hbm.at[pl.ds(base + c0 * t_chunk, t_chunk)], sems_o[0])
            wait_in(c1, 1)
            add_chunk(1)
            out1 = pltpu.async_copy(
                srcs[1], out_hbm.at[pl.ds(base + c1 * t_chunk, t_chunk)], sems_o[1])
            out0.wait()

            @pl.when(g + 1 < n_pairs)
            def _():
                issue_in(c0 + 2, 0)

            out1.wait()

            @pl.when(g + 1 < n_pairs)
            def _():
                issue_in(c1 + 2, 1)

            return carry

        lax.fori_loop(0, n_pairs, pair_body, 0)

    return k


def kernel(src, dates, encoding):
    b, s, d = src.shape
    n = b * s
    src2 = src.reshape(n, d)
    r = dates[..., 0].astype(jnp.int32).reshape(n)
    c = dates[..., 1].astype(jnp.int32).reshape(n)
    table = encoding.reshape(-1, d)
    out = _build_sc_kernel(n, d, 16)(src2, r, c, table)
    return out.reshape(b, s, d)


# speedup vs baseline: 2.9109x; 1.3577x over previous
import functools

import jax
import jax.numpy as jnp
from jax import lax
from jax.experimental import pallas as pl
from jax.experimental.pallas import tpu as pltpu
from jax.experimental.pallas import tpu_sc as plsc

ROWS = 12
COLS = 31
LANES = 16
NBUF = 3


@functools.lru_cache(maxsize=None)
def _build_sc_kernel(n_tokens, d, t_chunk):
    info = plsc.get_sparse_core_info()
    nc, ns = info.num_cores, info.num_subcores
    nw = nc * ns
    per_w = n_tokens // nw
    n_chunks = per_w // t_chunk
    n_groups = n_chunks // NBUF
    n_tail = n_chunks - n_groups * NBUF
    n_ivec = per_w // LANES
    mesh = plsc.VectorSubcoreMesh(core_axis_name="c", subcore_axis_name="s")

    scratch = [
        pltpu.VMEM((per_w,), jnp.int32),
        pltpu.VMEM((per_w,), jnp.int32),
        pltpu.VMEM((per_w,), jnp.int32),
    ]
    scratch += [pltpu.VMEM((t_chunk, d), jnp.float32) for _ in range(NBUF)]
    scratch += [pltpu.VMEM((t_chunk, d), jnp.float32) for _ in range(NBUF)]
    scratch += [pltpu.SemaphoreType.DMA for _ in range(3 * NBUF)]

    @functools.partial(
        pl.kernel,
        mesh=mesh,
        out_type=jax.ShapeDtypeStruct((n_tokens, d), jnp.float32),
        scratch_types=scratch,
    )
    def k(src_hbm, r_hbm, c_hbm, table_hbm, out_hbm, r_v, c_v, idx_v, *bufs):
        srcs = bufs[0:NBUF]
        encs = bufs[NBUF:2 * NBUF]
        sems_s = bufs[2 * NBUF:2 * NBUF + NBUF]
        sems_g = bufs[3 * NBUF:3 * NBUF + NBUF]
        sems_o = bufs[4 * NBUF:4 * NBUF + NBUF]
        wid = lax.axis_index("s") * nc + lax.axis_index("c")
        base = wid * per_w

        pltpu.sync_copy(r_hbm.at[pl.ds(base, per_w)], r_v)
        pltpu.sync_copy(c_hbm.at[pl.ds(base, per_w)], c_v)

        def idx_body(u, carry):
            sl = pl.ds(u * LANES, LANES)
            rv = r_v[sl] - 1
            rv = jnp.where(rv < 0, rv + ROWS, rv)
            cv = c_v[sl] - 1
            cv = jnp.where(cv < 0, cv + COLS, cv)
            idx_v[sl] = rv * COLS + cv
            return carry

        lax.fori_loop(0, n_ivec, idx_body, 0)

        def in_copies(ci, m):
            off = base + ci * t_chunk
            cs = pltpu.make_async_copy(
                src_hbm.at[pl.ds(off, t_chunk)], srcs[m], sems_s[m])
            cg = pltpu.make_async_copy(
                table_hbm.at[idx_v.at[pl.ds(ci * t_chunk, t_chunk)]],
                encs[m], sems_g[m])
            return cs, cg

        def issue_in(ci, m):
            cs, cg = in_copies(ci, m)
            cs.start()
            cg.start()

        def wait_in(ci, m):
            cs, cg = in_copies(ci, m)
            cs.wait()
            cg.wait()

        def out_copy(ci, m):
            return pltpu.make_async_copy(
                srcs[m], out_hbm.at[pl.ds(base + ci * t_chunk, t_chunk)],
                sems_o[m])

        def add_chunk(m):
            def body(t, carry):
                for j in range(d // LANES):
                    sl = pl.ds(j * LANES, LANES)
                    plsc.addupdate(srcs[m].at[t, sl], encs[m][t, sl])
                return carry

            lax.fori_loop(0, t_chunk, body, 0)

        def step(ci, m):
            wait_in(ci, m)
            add_chunk(m)
            out_copy(ci, m).start()
            if isinstance(ci, int):
                if ci >= 1:
                    out_copy(ci - 1, (m - 1) % NBUF).wait()
                if ci + 2 < n_chunks:
                    issue_in(ci + 2, (m + 2) % NBUF)
                return

            @pl.when(ci >= 1)
            def _():
                out_copy(ci - 1, (m - 1) % NBUF).wait()

            @pl.when(ci + 2 < n_chunks)
            def _():
                issue_in(ci + 2, (m + 2) % NBUF)

        issue_in(0, 0)
        issue_in(1, 1)

        def group_body(g, carry):
            for m in range(NBUF):
                step(g * NBUF + m, m)
            return carry

        lax.fori_loop(0, n_groups, group_body, 0)
        for e in range(n_tail):
            step(n_groups * NBUF + e, e)
        last = n_chunks - 1
        out_copy(last, last % NBUF).wait()

    return k


def kernel(src, dates, encoding):
    b, s, d = src.shape
    n = b * s
    src2 = src.reshape(n, d)
    r = dates[..., 0].astype(jnp.int32).reshape(n)
    c = dates[..., 1].astype(jnp.int32).reshape(n)
    table = encoding.reshape(-1, d)
    out = _build_sc_kernel(n, d, 16)(src2, r, c, table)
    return out.reshape(b, s, d)
